# baseline (speedup 1.0000x reference)
import functools

import jax
import jax.numpy as jnp
from jax import lax
from jax.experimental import pallas as pl
from jax.experimental.pallas import tpu as pltpu
from jax.experimental.pallas import tpu_sc as plsc

_N, _E, _D, _O = 10000, 320000, 128, 64
_NP = 10240
_NC, _NS = 2, 16
_NW = _NC * _NS
_EPW = _E // _NW
_CH = 80
_NCHUNK = _EPW // _CH
_RPT = _NP // _NS


def _make_sc_spmm(compute_deg: bool):
    mesh = plsc.VectorSubcoreMesh(core_axis_name="c", subcore_axis_name="s")
    out_type = [jax.ShapeDtypeStruct((_NC, _NP, _D), jnp.float32)]
    if compute_deg:
        out_type.append(jax.ShapeDtypeStruct((_NC, _NP), jnp.float32))
    _ZR = 8
    scratch = [
        pltpu.VMEM((_EPW,), jnp.int32),
        pltpu.VMEM((_CH,), jnp.int32),
        pltpu.VMEM((_CH,), jnp.int32),
        pltpu.VMEM((_CH,), jnp.int32),
        pltpu.VMEM((_CH, _D), jnp.float32),
        pltpu.VMEM((_CH, _D), jnp.float32),
        pltpu.VMEM((_CH, _D), jnp.float32),
        pltpu.VMEM((_ZR, _D), jnp.float32),
        pltpu.VMEM((_CH,), jnp.float32),
        pltpu.VMEM_SHARED((_NP, _D), jnp.float32),
        pltpu.VMEM_SHARED((_NP,), jnp.float32),
        pltpu.SemaphoreType.DMA,
        pltpu.SemaphoreType.DMA,
        pltpu.SemaphoreType.DMA,
        pltpu.SemaphoreType.DMA,
        pltpu.SemaphoreType.DMA,
        pltpu.SemaphoreType.DMA,
        pltpu.SemaphoreType.DMA,
        pltpu.SemaphoreType.DMA,
    ]

    @functools.partial(pl.kernel, out_type=tuple(out_type), mesh=mesh,
                       scratch_types=scratch)
    def spmm(feat_hbm, e2_hbm, e3_hbm, *refs):
        if compute_deg:
            (acc_out, deg_out, srcbuf, dv0, dv1, dv2, rows0, rows1, rows2,
             zblk, onesv, acc_sh, deg_sh, gsem0, gsem1, gsem2, ssem0, ssem1,
             ssem2, zsem, isem) = refs
        else:
            (acc_out, srcbuf, dv0, dv1, dv2, rows0, rows1, rows2,
             zblk, onesv, acc_sh, deg_sh, gsem0, gsem1, gsem2, ssem0, ssem1,
             ssem2, zsem, isem) = refs
        dv = (dv0, dv1, dv2)
        rows = (rows0, rows1, rows2)
        gsem = (gsem0, gsem1, gsem2)
        ssem = (ssem0, ssem1, ssem2)
        c_ax = lax.axis_index("c")
        s = lax.axis_index("s")
        wid = s * _NC + c_ax

        pltpu.async_copy(e2_hbm.at[0, wid], srcbuf, isem)

        zero16 = jnp.zeros((16,), jnp.float32)
        for i in range(_ZR):
            for k in range(_D // 16):
                zblk[i, pl.ds(k * 16, 16)] = zero16
        ones16 = jnp.ones((16,), jnp.float32)
        for k in range(_CH // 16):
            onesv[pl.ds(k * 16, 16)] = ones16

        accbase = s * _RPT
        for j in range(_RPT // _ZR):
            pltpu.async_copy(zblk, acc_sh.at[pl.ds(accbase + j * _ZR, _ZR)],
                             zsem)
        if compute_deg:
            for j in range(_RPT // _D):
                pltpu.async_copy(zblk.at[0],
                                 deg_sh.at[pl.ds(accbase + j * _D, _D)], zsem)

        def sidx(c):
            return srcbuf.at[pl.ds(c * _CH, _CH)]

        dbase = (_NW + wid) * _NCHUNK

        def prefetch(c, b):
            pltpu.async_copy(e3_hbm.at[dbase + c], dv[b], gsem[b])
            pltpu.async_copy(feat_hbm.at[sidx(c)], rows[b], gsem[b])

        def gwait(c, b):
            pltpu.make_async_copy(e3_hbm.at[dbase + c], dv[b],
                                  gsem[b]).wait()
            pltpu.make_async_copy(feat_hbm.at[sidx(c)], rows[b],
                                  gsem[b]).wait()

        def sfire(b):
            if compute_deg:
                pltpu.async_copy(onesv, deg_sh.at[dv[b]], ssem[b], add=True)
            pltpu.async_copy(rows[b], acc_sh.at[dv[b]], ssem[b], add=True)

        def sdrain(b):
            if compute_deg:
                pltpu.make_async_copy(onesv, deg_sh.at[dv[b]],
                                      ssem[b]).wait()
            pltpu.make_async_copy(rows[b], acc_sh.at[dv[b]],
                                  ssem[b]).wait()

        def stage(c, b, drain_prev, issue_next):
            gwait(c, b)
            sfire(b)
            pb = (b + 2) % 3
            if drain_prev:
                sdrain(pb)
            if issue_next:
                prefetch(c + 2, pb)

        pltpu.make_async_copy(e2_hbm.at[0, wid], srcbuf, isem).wait()
        prefetch(0, 0)
        prefetch(1, 1)
        for j in range(_RPT // _ZR):
            pltpu.make_async_copy(
                zblk, acc_sh.at[pl.ds(accbase, _ZR)], zsem).wait()
        if compute_deg:
            for j in range(_RPT // _D):
                pltpu.make_async_copy(
                    zblk.at[0], deg_sh.at[pl.ds(accbase, _D)], zsem).wait()
        plsc.subcore_barrier()

        stage(0, 0, drain_prev=False, issue_next=True)
        stage(1, 1, drain_prev=True, issue_next=True)

        def tri_body(j, _):
            c = 3 * j + 2
            stage(c, 2, drain_prev=True, issue_next=True)
            stage(c + 1, 0, drain_prev=True, issue_next=True)
            stage(c + 2, 1, drain_prev=True, issue_next=True)
            return 0

        lax.fori_loop(0, (_NCHUNK - 5) // 3, tri_body, 0)
        stage(_NCHUNK - 3, 2, drain_prev=True, issue_next=True)
        stage(_NCHUNK - 2, 0, drain_prev=True, issue_next=False)
        stage(_NCHUNK - 1, 1, drain_prev=True, issue_next=False)
        sdrain((_NCHUNK - 1) % 3)
        plsc.subcore_barrier()

        pltpu.sync_copy(acc_sh.at[pl.ds(accbase, _RPT)],
                        acc_out.at[c_ax, pl.ds(accbase, _RPT)])
        if compute_deg:
            pltpu.sync_copy(deg_sh.at[pl.ds(accbase, _RPT)],
                            deg_out.at[c_ax, pl.ds(accbase, _RPT)])

    return spmm


_make_sc_spmm = functools.lru_cache(maxsize=None)(_make_sc_spmm)

_BR = 1000
_GRID = _N // _BR


def _tc_self_body(x_ref, ws_ref, b_ref, o_ref):
    o_ref[...] = x_ref[...] @ ws_ref[...] + b_ref[...]


def _tc_layer1_body(xs_ref, acc_ref, deg_ref, wn_ref, ws2_ref, b2_ref,
                    o_ref, os_ref):
    deg = deg_ref[0] + deg_ref[1]
    invd = 1.0 / jnp.maximum(deg, 1.0)
    m = (acc_ref[0] + acc_ref[1]) * invd
    h1 = jnp.maximum(xs_ref[...] + m @ wn_ref[...], 0.0)
    o_ref[...] = h1
    os_ref[...] = h1 @ ws2_ref[...] + b2_ref[...]


def _tc_head_body(hs_ref, acc_ref, deg_ref, wn_ref,
                  wm1_ref, bm1_ref, wm2_ref, bm2_ref, o_ref):
    deg = deg_ref[0] + deg_ref[1]
    invd = 1.0 / jnp.maximum(deg, 1.0)
    m = (acc_ref[0] + acc_ref[1]) * invd
    h2 = jnp.maximum(hs_ref[...] + m @ wn_ref[...], 0.0)
    h3 = jnp.maximum(h2 @ wm1_ref[...] + bm1_ref[...], 0.0)
    o_ref[...] = h3 @ wm2_ref[...] + bm2_ref[...]


_row_spec = pl.BlockSpec((_BR, _D), lambda r: (r, 0))
_acc_spec = pl.BlockSpec((_NC, _BR, _D), lambda r: (0, r, 0))
_deg_spec = pl.BlockSpec((_NC, _BR, 1), lambda r: (0, r, 0))
_w_spec = pl.BlockSpec((_D, _D), lambda r: (0, 0))
_b_spec = pl.BlockSpec((1, _D), lambda r: (0, 0))

_tc_self = pl.pallas_call(
    _tc_self_body,
    grid=(_GRID,),
    in_specs=[_row_spec, _w_spec, _b_spec],
    out_specs=_row_spec,
    out_shape=jax.ShapeDtypeStruct((_N, _D), jnp.float32),
)

_tc_layer1 = pl.pallas_call(
    _tc_layer1_body,
    grid=(_GRID,),
    in_specs=[_row_spec, _acc_spec, _deg_spec, _w_spec, _w_spec, _b_spec],
    out_specs=[_row_spec, _row_spec],
    out_shape=[jax.ShapeDtypeStruct((_N, _D), jnp.float32),
               jax.ShapeDtypeStruct((_N, _D), jnp.float32)],
)

_tc_head = pl.pallas_call(
    _tc_head_body,
    grid=(_GRID,),
    in_specs=[_row_spec, _acc_spec, _deg_spec, _w_spec,
              _w_spec, _b_spec,
              pl.BlockSpec((_D, _O), lambda r: (0, 0)),
              pl.BlockSpec((1, _O), lambda r: (0, 0))],
    out_specs=pl.BlockSpec((_BR, _O), lambda r: (r, 0)),
    out_shape=jax.ShapeDtypeStruct((_N, _O), jnp.float32),
)


def kernel(x, edge_index, W_self1, W_neigh1, b1, W_self2, W_neigh2, b2,
           Wm1, bm1, Wm2, bm2):
    e2 = edge_index.reshape(2, _NW, _EPW)
    e3 = edge_index.reshape(2 * _NW * _NCHUNK, _CH)

    acc1, degp = _make_sc_spmm(True)(x, e2, e3)
    xs = _tc_self(x, W_self1, b1.reshape(1, _D))
    deg3 = degp.reshape(_NC, _NP, 1)
    h1, hs = _tc_layer1(xs, acc1, deg3, W_neigh1, W_self2,
                        b2.reshape(1, _D))

    (acc2,) = _make_sc_spmm(False)(h1, e2, e3)
    out = _tc_head(hs, acc2, deg3, W_neigh2,
                   Wm1, bm1.reshape(1, _D), Wm2, bm2.reshape(1, _O))
    return out


# speedup vs baseline: 1.0058x; 1.0058x over previous
import functools

import jax
import jax.numpy as jnp
from jax import lax
from jax.experimental import pallas as pl
from jax.experimental.pallas import tpu as pltpu
from jax.experimental.pallas import tpu_sc as plsc

_N, _E, _D, _O = 10000, 320000, 128, 64
_NP = 10240
_NC, _NS = 2, 16
_NW = _NC * _NS
_EPW = _E // _NW
_CH = 80
_NCHUNK = _EPW // _CH
_RPT = _NP // _NS


def _make_sc_spmm(compute_deg: bool):
    mesh = plsc.VectorSubcoreMesh(core_axis_name="c", subcore_axis_name="s")
    out_type = [jax.ShapeDtypeStruct((_NC, _NP, _D), jnp.float32)]
    if compute_deg:
        out_type.append(jax.ShapeDtypeStruct((_NC, _NP), jnp.float32))
    _ZR = 40
    scratch = [
        pltpu.VMEM((_EPW,), jnp.int32),
        pltpu.VMEM((_CH,), jnp.int32),
        pltpu.VMEM((_CH,), jnp.int32),
        pltpu.VMEM((_CH,), jnp.int32),
        pltpu.VMEM((_CH, _D), jnp.float32),
        pltpu.VMEM((_CH, _D), jnp.float32),
        pltpu.VMEM((_CH, _D), jnp.float32),
        pltpu.VMEM((_ZR, _D), jnp.float32),
        pltpu.VMEM((_CH,), jnp.float32),
        pltpu.VMEM_SHARED((_NP, _D), jnp.float32),
        pltpu.VMEM_SHARED((_NP,), jnp.float32),
        pltpu.SemaphoreType.DMA,
        pltpu.SemaphoreType.DMA,
        pltpu.SemaphoreType.DMA,
        pltpu.SemaphoreType.DMA,
        pltpu.SemaphoreType.DMA,
        pltpu.SemaphoreType.DMA,
        pltpu.SemaphoreType.DMA,
        pltpu.SemaphoreType.DMA,
    ]

    @functools.partial(pl.kernel, out_type=tuple(out_type), mesh=mesh,
                       scratch_types=scratch)
    def spmm(feat_hbm, e2_hbm, e3_hbm, *refs):
        if compute_deg:
            (acc_out, deg_out, srcbuf, dv0, dv1, dv2, rows0, rows1, rows2,
             zblk, onesv, acc_sh, deg_sh, gsem0, gsem1, gsem2, ssem0, ssem1,
             ssem2, zsem, isem) = refs
        else:
            (acc_out, srcbuf, dv0, dv1, dv2, rows0, rows1, rows2,
             zblk, onesv, acc_sh, deg_sh, gsem0, gsem1, gsem2, ssem0, ssem1,
             ssem2, zsem, isem) = refs
        dv = (dv0, dv1, dv2)
        rows = (rows0, rows1, rows2)
        gsem = (gsem0, gsem1, gsem2)
        ssem = (ssem0, ssem1, ssem2)
        c_ax = lax.axis_index("c")
        s = lax.axis_index("s")
        wid = s * _NC + c_ax

        pltpu.async_copy(e2_hbm.at[0, wid], srcbuf, isem)

        zero16 = jnp.zeros((16,), jnp.float32)
        for i in range(_ZR):
            for k in range(_D // 16):
                zblk[i, pl.ds(k * 16, 16)] = zero16
        ones16 = jnp.ones((16,), jnp.float32)
        for k in range(_CH // 16):
            onesv[pl.ds(k * 16, 16)] = ones16

        accbase = s * _RPT
        for j in range(_RPT // _ZR):
            pltpu.async_copy(zblk, acc_sh.at[pl.ds(accbase + j * _ZR, _ZR)],
                             zsem)
        if compute_deg:
            for j in range(_RPT // _D):
                pltpu.async_copy(zblk.at[0],
                                 deg_sh.at[pl.ds(accbase + j * _D, _D)], zsem)

        def sidx(c):
            return srcbuf.at[pl.ds(c * _CH, _CH)]

        dbase = (_NW + wid) * _NCHUNK

        def prefetch(c, b):
            pltpu.async_copy(e3_hbm.at[dbase + c], dv[b], gsem[b])
            pltpu.async_copy(feat_hbm.at[sidx(c)], rows[b], gsem[b])

        def gwait(c, b):
            pltpu.make_async_copy(e3_hbm.at[dbase + c], dv[b],
                                  gsem[b]).wait()
            pltpu.make_async_copy(feat_hbm.at[sidx(c)], rows[b],
                                  gsem[b]).wait()

        def sfire(b):
            if compute_deg:
                pltpu.async_copy(onesv, deg_sh.at[dv[b]], ssem[b], add=True)
            pltpu.async_copy(rows[b], acc_sh.at[dv[b]], ssem[b], add=True)

        def sdrain(b):
            if compute_deg:
                pltpu.make_async_copy(onesv, deg_sh.at[dv[b]],
                                      ssem[b]).wait()
            pltpu.make_async_copy(rows[b], acc_sh.at[dv[b]],
                                  ssem[b]).wait()

        def stage(c, b, drain_prev, issue_next):
            gwait(c, b)
            sfire(b)
            pb = (b + 2) % 3
            if drain_prev:
                sdrain(pb)
            if issue_next:
                prefetch(c + 2, pb)

        pltpu.make_async_copy(e2_hbm.at[0, wid], srcbuf, isem).wait()
        prefetch(0, 0)
        prefetch(1, 1)
        for j in range(_RPT // _ZR):
            pltpu.make_async_copy(
                zblk, acc_sh.at[pl.ds(accbase, _ZR)], zsem).wait()
        if compute_deg:
            for j in range(_RPT // _D):
                pltpu.make_async_copy(
                    zblk.at[0], deg_sh.at[pl.ds(accbase, _D)], zsem).wait()
        plsc.subcore_barrier()

        stage(0, 0, drain_prev=False, issue_next=True)
        stage(1, 1, drain_prev=True, issue_next=True)

        def tri_body(j, _):
            c = 3 * j + 2
            stage(c, 2, drain_prev=True, issue_next=True)
            stage(c + 1, 0, drain_prev=True, issue_next=True)
            stage(c + 2, 1, drain_prev=True, issue_next=True)
            return 0

        lax.fori_loop(0, (_NCHUNK - 5) // 3, tri_body, 0)
        stage(_NCHUNK - 3, 2, drain_prev=True, issue_next=True)
        stage(_NCHUNK - 2, 0, drain_prev=True, issue_next=False)
        stage(_NCHUNK - 1, 1, drain_prev=True, issue_next=False)
        sdrain((_NCHUNK - 1) % 3)
        plsc.subcore_barrier()

        wb = pltpu.async_copy(acc_sh.at[pl.ds(accbase, _RPT)],
                              acc_out.at[c_ax, pl.ds(accbase, _RPT)], zsem)
        if compute_deg:
            pltpu.async_copy(deg_sh.at[pl.ds(accbase, _RPT)],
                             deg_out.at[c_ax, pl.ds(accbase, _RPT)], isem)
        wb.wait()
        if compute_deg:
            pltpu.make_async_copy(
                deg_sh.at[pl.ds(accbase, _RPT)],
                deg_out.at[c_ax, pl.ds(accbase, _RPT)], isem).wait()

    return spmm


_make_sc_spmm = functools.lru_cache(maxsize=None)(_make_sc_spmm)

_BR = 1000
_GRID = _N // _BR


def _tc_self_body(x_ref, ws_ref, b_ref, o_ref):
    o_ref[...] = x_ref[...] @ ws_ref[...] + b_ref[...]


def _tc_layer1_body(xs_ref, acc_ref, deg_ref, wn_ref, ws2_ref, b2_ref,
                    o_ref, os_ref):
    deg = deg_ref[0] + deg_ref[1]
    invd = 1.0 / jnp.maximum(deg, 1.0)
    m = (acc_ref[0] + acc_ref[1]) * invd
    h1 = jnp.maximum(xs_ref[...] + m @ wn_ref[...], 0.0)
    o_ref[...] = h1
    os_ref[...] = h1 @ ws2_ref[...] + b2_ref[...]


def _tc_head_body(hs_ref, acc_ref, deg_ref, wn_ref,
                  wm1_ref, bm1_ref, wm2_ref, bm2_ref, o_ref):
    deg = deg_ref[0] + deg_ref[1]
    invd = 1.0 / jnp.maximum(deg, 1.0)
    m = (acc_ref[0] + acc_ref[1]) * invd
    h2 = jnp.maximum(hs_ref[...] + m @ wn_ref[...], 0.0)
    h3 = jnp.maximum(h2 @ wm1_ref[...] + bm1_ref[...], 0.0)
    o_ref[...] = h3 @ wm2_ref[...] + bm2_ref[...]


_row_spec = pl.BlockSpec((_BR, _D), lambda r: (r, 0))
_acc_spec = pl.BlockSpec((_NC, _BR, _D), lambda r: (0, r, 0))
_deg_spec = pl.BlockSpec((_NC, _BR, 1), lambda r: (0, r, 0))
_w_spec = pl.BlockSpec((_D, _D), lambda r: (0, 0))
_b_spec = pl.BlockSpec((1, _D), lambda r: (0, 0))

_tc_self = pl.pallas_call(
    _tc_self_body,
    grid=(_GRID,),
    in_specs=[_row_spec, _w_spec, _b_spec],
    out_specs=_row_spec,
    out_shape=jax.ShapeDtypeStruct((_N, _D), jnp.float32),
)

_tc_layer1 = pl.pallas_call(
    _tc_layer1_body,
    grid=(_GRID,),
    in_specs=[_row_spec, _acc_spec, _deg_spec, _w_spec, _w_spec, _b_spec],
    out_specs=[_row_spec, _row_spec],
    out_shape=[jax.ShapeDtypeStruct((_N, _D), jnp.float32),
               jax.ShapeDtypeStruct((_N, _D), jnp.float32)],
)

_tc_head = pl.pallas_call(
    _tc_head_body,
    grid=(_GRID,),
    in_specs=[_row_spec, _acc_spec, _deg_spec, _w_spec,
              _w_spec, _b_spec,
              pl.BlockSpec((_D, _O), lambda r: (0, 0)),
              pl.BlockSpec((1, _O), lambda r: (0, 0))],
    out_specs=pl.BlockSpec((_BR, _O), lambda r: (r, 0)),
    out_shape=jax.ShapeDtypeStruct((_N, _O), jnp.float32),
)


def kernel(x, edge_index, W_self1, W_neigh1, b1, W_self2, W_neigh2, b2,
           Wm1, bm1, Wm2, bm2):
    e2 = edge_index.reshape(2, _NW, _EPW)
    e3 = edge_index.reshape(2 * _NW * _NCHUNK, _CH)

    acc1, degp = _make_sc_spmm(True)(x, e2, e3)
    xs = _tc_self(x, W_self1, b1.reshape(1, _D))
    deg3 = degp.reshape(_NC, _NP, 1)
    h1, hs = _tc_layer1(xs, acc1, deg3, W_neigh1, W_self2,
                        b2.reshape(1, _D))

    (acc2,) = _make_sc_spmm(False)(h1, e2, e3)
    out = _tc_head(hs, acc2, deg3, W_neigh2,
                   Wm1, bm1.reshape(1, _D), Wm2, bm2.reshape(1, _O))
    return out
